# Initial kernel scaffold; baseline (speedup 1.0000x reference)
import jax
import jax.numpy as jnp
from jax.experimental import pallas as pl


def kernel(vals, edges, pseudo, multi_gpu, Wg0, mu0, sigma0, root0, bias0, Wd0, Wg1, mu1, sigma1, root1, bias1, Wd1):
    raise NotImplementedError("write your pallas kernel here")


# speedup vs baseline: 1.1547x; 1.1547x over previous
import functools

import jax
import jax.numpy as jnp
from jax import lax
from jax.experimental import pallas as pl
from jax.experimental.pallas import tpu as pltpu
from jax.experimental.pallas import tpu_sc as plsc

N = 10000
NE = 320000
K = 3
C = 128
KC = K * C
EPS = 1e-15

NC = 2
NS = 16
NW = NC * NS
EPT = NE // NW
CHUNK = 16
NCHUNK = EPT // CHUNK
RMAIN = 624
RTAIL = N - NS * RMAIN
ZROWS = 48
CW = C

BLK = 1000


def _edge_kernel_fn(compute_deg):
    mesh = plsc.VectorSubcoreMesh(core_axis_name="c", subcore_axis_name="s")

    @functools.partial(
        pl.kernel,
        mesh=mesh,
        out_type=jax.ShapeDtypeStruct((NC, N, CW), jnp.float32),
        scratch_types=[
            pltpu.VMEM((CHUNK,), jnp.int32),
            pltpu.VMEM((CHUNK,), jnp.int32),
            pltpu.VMEM((CHUNK, C), jnp.float32),
            pltpu.VMEM((CHUNK, C), jnp.float32),
            pltpu.VMEM((CHUNK, C), jnp.float32),
            pltpu.VMEM((CHUNK, CW), jnp.float32),
            pltpu.VMEM((CHUNK,), jnp.float32),
            pltpu.VMEM((CHUNK,), jnp.float32),
            pltpu.VMEM((K, CHUNK), jnp.float32),
            pltpu.VMEM((256,), jnp.float32),
            pltpu.VMEM((ZROWS, CW), jnp.float32),
            pltpu.VMEM((ZROWS,), jnp.int32),
            pltpu.SemaphoreType.DMA,
            pltpu.SemaphoreType.DMA,
            pltpu.SemaphoreType.DMA,
            pltpu.VMEM_SHARED((N, CW), jnp.float32),
        ],
    )
    def edge_kernel(xg0_hbm, xg1_hbm, xg2_hbm, src_hbm, dst_hbm,
                    p0_hbm, p1_hbm, coef_hbm,
                    agg_out,
                    srcb, dstb, rows0, rows1, rows2, msgw, p0b, p1b, gb,
                    coefb, stg, zidx,
                    sem0, sem1, sem2, agg_sh):
        cc = lax.axis_index("c")
        ss = lax.axis_index("s")
        wid = ss * NC + cc

        pltpu.sync_copy(coef_hbm, coefb)

        def zrow(q, zcarry):
            for j in range(CW // 16):
                stg[q, pl.ds(j * 16, 16)] = jnp.zeros((16,), jnp.float32)
            return zcarry

        lax.fori_loop(0, ZROWS, zrow, 0)

        r0 = ss * RMAIN

        def zcopy(q, zcarry):
            b0 = r0 + q * ZROWS
            for v in range(ZROWS // 16):
                zidx[pl.ds(v * 16, 16)] = (lax.iota(jnp.int32, 16)
                                           + (b0 + v * 16))
            pltpu.sync_copy(stg, agg_sh.at[zidx])
            return zcarry

        lax.fori_loop(0, RMAIN // ZROWS, zcopy, 0)

        @pl.when(ss == NS - 1)
        def _():
            zidx[pl.ds(0, 16)] = lax.iota(jnp.int32, 16) + (NS * RMAIN)
            zidx[pl.ds(16, 16)] = lax.iota(jnp.int32, 16)
            zidx[pl.ds(32, 16)] = lax.iota(jnp.int32, 16) + 16
            pltpu.sync_copy(stg, agg_sh.at[zidx])

        plsc.subcore_barrier()

        base = wid * EPT

        def chunk_body(i, carry):
            off = base + i * CHUNK
            pltpu.sync_copy(src_hbm.at[pl.ds(off, CHUNK)], srcb)
            pltpu.sync_copy(dst_hbm.at[pl.ds(off, CHUNK)], dstb)
            pltpu.sync_copy(p0_hbm.at[pl.ds(off, CHUNK)], p0b)
            pltpu.sync_copy(p1_hbm.at[pl.ds(off, CHUNK)], p1b)
            cp0 = pltpu.async_copy(xg0_hbm.at[srcb], rows0, sem0)
            cp1 = pltpu.async_copy(xg1_hbm.at[srcb], rows1, sem1)
            cp2 = pltpu.async_copy(xg2_hbm.at[srcb], rows2, sem2)
            cp0.wait()
            cp1.wait()
            cp2.wait()

            for v in range(CHUNK // 16):
                sl = pl.ds(v * 16, 16)
                p0v = p0b[sl]
                p1v = p1b[sl]
                for k in range(K):
                    c0 = coefb[pl.ds((5 * k + 0) * 16, 16)]
                    c1 = coefb[pl.ds((5 * k + 1) * 16, 16)]
                    c2 = coefb[pl.ds((5 * k + 2) * 16, 16)]
                    c3 = coefb[pl.ds((5 * k + 3) * 16, 16)]
                    c4 = coefb[pl.ds((5 * k + 4) * 16, 16)]
                    gb[k, sl] = jnp.exp(c0 + p0v * (c1 + c2 * p0v)
                                        + p1v * (c3 + c4 * p1v))

            def group_body(gi, gcarry):
                sl = pl.ds(gi * 16, 16)
                g0v = gb[0, sl]
                g1v = gb[1, sl]
                g2v = gb[2, sl]
                e0 = gi * 16
                for lane in range(16):
                    e = e0 + lane
                    g0 = g0v[lane]
                    g1 = g1v[lane]
                    g2 = g2v[lane]
                    for j in range(C // 16):
                        slj = pl.ds(j * 16, 16)
                        a = rows0[e, slj]
                        b = rows1[e, slj]
                        c = rows2[e, slj]
                        msgw[e, slj] = g0 * a + g1 * b + g2 * c
                return gcarry

            lax.fori_loop(0, CHUNK // 16, group_body, 0)

            pltpu.sync_copy(msgw, agg_sh.at[dstb], add=True)
            return carry

        lax.fori_loop(0, NCHUNK, chunk_body, 0)

        plsc.subcore_barrier()

        def ocopy(q, ocarry):
            b0 = r0 + q * ZROWS
            for v in range(ZROWS // 16):
                zidx[pl.ds(v * 16, 16)] = (lax.iota(jnp.int32, 16)
                                           + (b0 + v * 16))
            pltpu.sync_copy(agg_sh.at[zidx], stg)
            pltpu.sync_copy(stg, agg_out.at[cc, pl.ds(b0, ZROWS)])
            return ocarry

        lax.fori_loop(0, RMAIN // ZROWS, ocopy, 0)

        @pl.when(ss == NS - 1)
        def _():
            zidx[pl.ds(0, 16)] = lax.iota(jnp.int32, 16) + (NS * RMAIN)
            zidx[pl.ds(16, 16)] = lax.iota(jnp.int32, 16)
            zidx[pl.ds(32, 16)] = lax.iota(jnp.int32, 16) + 16
            pltpu.sync_copy(agg_sh.at[zidx], stg)
            pltpu.sync_copy(stg.at[pl.ds(0, RTAIL)],
                            agg_out.at[cc, pl.ds(NS * RMAIN, RTAIL)])

    return edge_kernel


_edge_pass = _edge_kernel_fn(True)


def _deg_kernel_fn():
    mesh = plsc.VectorSubcoreMesh(core_axis_name="c", subcore_axis_name="s")

    @functools.partial(
        pl.kernel,
        mesh=mesh,
        out_type=jax.ShapeDtypeStruct((NC, N, C), jnp.float32),
        scratch_types=[
            pltpu.VMEM((CHUNK,), jnp.int32),
            pltpu.VMEM((CHUNK, C), jnp.float32),
            pltpu.VMEM((ZROWS, C), jnp.float32),
            pltpu.VMEM((ZROWS,), jnp.int32),
            pltpu.VMEM_SHARED((N, C), jnp.float32),
        ],
    )
    def deg_kernel(dst_hbm, deg_out, dstb, onesb, stg, zidx, deg_sh):
        cc = lax.axis_index("c")
        ss = lax.axis_index("s")
        wid = ss * NC + cc

        def zrow(q, zcarry):
            for j in range(C // 16):
                stg[q, pl.ds(j * 16, 16)] = jnp.zeros((16,), jnp.float32)
            return zcarry

        lax.fori_loop(0, ZROWS, zrow, 0)

        def onerow(q, ocarry):
            for j in range(C // 16):
                onesb[q, pl.ds(j * 16, 16)] = (jnp.zeros((16,), jnp.float32)
                                               + 1.0)
            return ocarry

        lax.fori_loop(0, CHUNK, onerow, 0)

        r0 = ss * RMAIN

        def zcopy(q, zcarry):
            b0 = r0 + q * ZROWS
            for v in range(ZROWS // 16):
                zidx[pl.ds(v * 16, 16)] = (lax.iota(jnp.int32, 16)
                                           + (b0 + v * 16))
            pltpu.sync_copy(stg, deg_sh.at[zidx])
            return zcarry

        lax.fori_loop(0, RMAIN // ZROWS, zcopy, 0)

        @pl.when(ss == NS - 1)
        def _():
            zidx[pl.ds(0, 16)] = lax.iota(jnp.int32, 16) + (NS * RMAIN)
            zidx[pl.ds(16, 16)] = lax.iota(jnp.int32, 16)
            zidx[pl.ds(32, 16)] = lax.iota(jnp.int32, 16) + 16
            pltpu.sync_copy(stg, deg_sh.at[zidx])

        plsc.subcore_barrier()

        base = wid * EPT

        def chunk_body(i, carry):
            off = base + i * CHUNK
            pltpu.sync_copy(dst_hbm.at[pl.ds(off, CHUNK)], dstb)
            pltpu.sync_copy(onesb, deg_sh.at[dstb], add=True)
            return carry

        lax.fori_loop(0, NCHUNK, chunk_body, 0)

        plsc.subcore_barrier()

        def ocopy(q, ocarry):
            b0 = r0 + q * ZROWS
            for v in range(ZROWS // 16):
                zidx[pl.ds(v * 16, 16)] = (lax.iota(jnp.int32, 16)
                                           + (b0 + v * 16))
            pltpu.sync_copy(deg_sh.at[zidx], stg)
            pltpu.sync_copy(stg, deg_out.at[cc, pl.ds(b0, ZROWS)])
            return ocarry

        lax.fori_loop(0, RMAIN // ZROWS, ocopy, 0)

        @pl.when(ss == NS - 1)
        def _():
            zidx[pl.ds(0, 16)] = lax.iota(jnp.int32, 16) + (NS * RMAIN)
            zidx[pl.ds(16, 16)] = lax.iota(jnp.int32, 16)
            zidx[pl.ds(32, 16)] = lax.iota(jnp.int32, 16) + 16
            pltpu.sync_copy(deg_sh.at[zidx], stg)
            pltpu.sync_copy(stg.at[pl.ds(0, RTAIL)],
                            deg_out.at[cc, pl.ds(NS * RMAIN, RTAIL)])

    return deg_kernel


_deg_pass = _deg_kernel_fn()


def _coef_mat(mu, sigma):
    a = -0.5 / (EPS + sigma.astype(jnp.float32) ** 2)
    c0 = a[:, 0] * mu[:, 0] ** 2 + a[:, 1] * mu[:, 1] ** 2
    c1 = -2.0 * a[:, 0] * mu[:, 0]
    c2 = a[:, 0]
    c3 = -2.0 * a[:, 1] * mu[:, 1]
    c4 = a[:, 1]
    co = jnp.stack([c0, c1, c2, c3, c4], axis=1).reshape(-1)
    co = jnp.concatenate([co, jnp.zeros((1,), jnp.float32)])
    return jnp.broadcast_to(co[:, None], (16, 16)).reshape(-1)


def _pre_body(x_ref, wg_ref, root_ref, wd_ref, b_ref, xg0_ref, xg1_ref,
              xg2_ref, dense_ref):
    xg_refs = (xg0_ref, xg1_ref, xg2_ref)
    x = x_ref[...]
    wg = wg_ref[...]
    for k in range(K):
        xg_refs[k][...] = jnp.dot(x, wg[:, k * C:(k + 1) * C],
                                  preferred_element_type=jnp.float32)
    dense_ref[...] = (jnp.dot(x, root_ref[...] + wd_ref[...],
                              preferred_element_type=jnp.float32) + b_ref[...])


def _pre(x, Wg, root, Wd, bias):
    return pl.pallas_call(
        _pre_body,
        grid=(N // BLK,),
        in_specs=[
            pl.BlockSpec((BLK, C), lambda i: (i, 0)),
            pl.BlockSpec((C, KC), lambda i: (0, 0)),
            pl.BlockSpec((C, C), lambda i: (0, 0)),
            pl.BlockSpec((C, C), lambda i: (0, 0)),
            pl.BlockSpec((1, C), lambda i: (0, 0)),
        ],
        out_specs=[
            pl.BlockSpec((BLK, C), lambda i: (i, 0)),
            pl.BlockSpec((BLK, C), lambda i: (i, 0)),
            pl.BlockSpec((BLK, C), lambda i: (i, 0)),
            pl.BlockSpec((BLK, C), lambda i: (i, 0)),
        ],
        out_shape=[
            jax.ShapeDtypeStruct((N, C), jnp.float32),
            jax.ShapeDtypeStruct((N, C), jnp.float32),
            jax.ShapeDtypeStruct((N, C), jnp.float32),
            jax.ShapeDtypeStruct((N, C), jnp.float32),
        ],
    )(x, Wg, root, Wd, bias.reshape(1, C))


def _combine(aggp_ref, degp_ref, dense_ref):
    agg = aggp_ref[0] + aggp_ref[1]
    deg = degp_ref[0, :, 0:1] + degp_ref[1, :, 0:1]
    return agg / jnp.clip(deg, 1.0, None) + dense_ref[...]


def _mid_body(aggp_ref, degp_ref, dense_ref, wg_ref, root_ref, wd_ref, b_ref,
              xg0_ref, xg1_ref, xg2_ref, dense_out_ref):
    xg_refs = (xg0_ref, xg1_ref, xg2_ref)
    h = jnp.maximum(_combine(aggp_ref, degp_ref, dense_ref), 0.0)
    wg = wg_ref[...]
    for k in range(K):
        xg_refs[k][...] = jnp.dot(h, wg[:, k * C:(k + 1) * C],
                                  preferred_element_type=jnp.float32)
    dense_out_ref[...] = (jnp.dot(h, root_ref[...] + wd_ref[...],
                                  preferred_element_type=jnp.float32) + b_ref[...])


def _mid(aggp, degp, dense, Wg, root, Wd, bias):
    return pl.pallas_call(
        _mid_body,
        grid=(N // BLK,),
        in_specs=[
            pl.BlockSpec((NC, BLK, CW), lambda i: (0, i, 0)),
            pl.BlockSpec((NC, BLK, C), lambda i: (0, i, 0)),
            pl.BlockSpec((BLK, C), lambda i: (i, 0)),
            pl.BlockSpec((C, KC), lambda i: (0, 0)),
            pl.BlockSpec((C, C), lambda i: (0, 0)),
            pl.BlockSpec((C, C), lambda i: (0, 0)),
            pl.BlockSpec((1, C), lambda i: (0, 0)),
        ],
        out_specs=[
            pl.BlockSpec((BLK, C), lambda i: (i, 0)),
            pl.BlockSpec((BLK, C), lambda i: (i, 0)),
            pl.BlockSpec((BLK, C), lambda i: (i, 0)),
            pl.BlockSpec((BLK, C), lambda i: (i, 0)),
        ],
        out_shape=[
            jax.ShapeDtypeStruct((N, C), jnp.float32),
            jax.ShapeDtypeStruct((N, C), jnp.float32),
            jax.ShapeDtypeStruct((N, C), jnp.float32),
            jax.ShapeDtypeStruct((N, C), jnp.float32),
        ],
    )(aggp, degp, dense, Wg, root, Wd, bias.reshape(1, C))


def _post_body(aggp_ref, degp_ref, dense_ref, out_ref):
    out_ref[...] = _combine(aggp_ref, degp_ref, dense_ref)


def _post(aggp, degp, dense):
    return pl.pallas_call(
        _post_body,
        grid=(N // BLK,),
        in_specs=[
            pl.BlockSpec((NC, BLK, CW), lambda i: (0, i, 0)),
            pl.BlockSpec((NC, BLK, C), lambda i: (0, i, 0)),
            pl.BlockSpec((BLK, C), lambda i: (i, 0)),
        ],
        out_specs=pl.BlockSpec((BLK, C), lambda i: (i, 0)),
        out_shape=jax.ShapeDtypeStruct((N, C), jnp.float32),
    )(aggp, degp, dense)


def kernel(vals, edges, pseudo, multi_gpu, Wg0, mu0, sigma0, root0, bias0, Wd0,
           Wg1, mu1, sigma1, root1, bias1, Wd1):
    src = edges[0]
    dst = edges[1]
    p0 = pseudo[:, 0] + 0.0
    p1 = pseudo[:, 1] + 0.0
    coef0 = _coef_mat(mu0, sigma0)
    coef1 = _coef_mat(mu1, sigma1)

    degp = _deg_pass(dst)
    a0, b0, c0, dense0 = _pre(vals, Wg0, root0, Wd0, bias0)
    agg0 = _edge_pass(a0, b0, c0, src, dst, p0, p1, coef0)
    a1, b1, c1, dense1 = _mid(agg0, degp, dense0, Wg1, root1, Wd1, bias1)
    agg1 = _edge_pass(a1, b1, c1, src, dst, p0, p1, coef1)
    return _post(agg1, degp, dense1)
